# baseline (speedup 1.0000x reference)
import functools
import math

import jax
import jax.numpy as jnp
from jax import lax
from jax.experimental import pallas as pl
from jax.experimental.pallas import tpu as pltpu
from jax.experimental.pallas import tpu_sc as plsc

N_NODES = 10000
D_IN = 128
D_OUT = 128
DH = 64
N_EDGES = 320000
EPS = 1e-5

N_TILES = 16
CHUNK = 128
NCH = 158
E_PAD = NCH * N_TILES * CHUNK
EDGES_PER_TILE = NCH * CHUNK

BN = 1000
N_PAD = 10240
ROWS_PER_TILE = N_PAD // N_TILES

_INV_SQRT2 = 1.0 / math.sqrt(2.0)


def _dense_body(x_ref, wt_ref, b_ref, g_ref, be_ref, o_ref):
    h = lax.dot_general(
        x_ref[...], wt_ref[...], (((1,), (0,)), ((), ())),
        preferred_element_type=jnp.float32,
    )
    h = h + b_ref[...]
    m = jnp.mean(h, axis=1, keepdims=True)
    d = h - m
    v = jnp.mean(d * d, axis=1, keepdims=True)
    hn = d * lax.rsqrt(v + EPS)
    hn = hn * g_ref[...] + be_ref[...]
    g = 0.5 * hn * (1.0 + lax.erf(hn * _INV_SQRT2))
    o_ref[0] = g[:, :DH]
    o_ref[1] = g[:, DH:]


def _dense(x2, wt, b2, g2, be2):
    return pl.pallas_call(
        _dense_body,
        grid=(N_NODES // BN,),
        in_specs=[
            pl.BlockSpec((BN, D_IN), lambda i: (i, 0)),
            pl.BlockSpec((D_IN, D_OUT), lambda i: (0, 0)),
            pl.BlockSpec((1, D_OUT), lambda i: (0, 0)),
            pl.BlockSpec((BN, 1), lambda i: (i, 0)),
            pl.BlockSpec((BN, 1), lambda i: (i, 0)),
        ],
        out_specs=pl.BlockSpec((2, BN, DH), lambda i: (0, i, 0)),
        out_shape=jax.ShapeDtypeStruct((2, N_NODES, DH), jnp.float32),
    )(x2, wt, b2, g2, be2)


def _mp_body(h2, idxpack, normpack, zeros, out,
             idx_v, norms_v, rows0_v, rows1_v, acc_sh,
             sg0, sg1, ss0, ss1):
    c = lax.axis_index("c")
    s = lax.axis_index("s")
    pltpu.sync_copy(idxpack.at[s], idx_v)
    pltpu.sync_copy(normpack.at[s], norms_v)
    pltpu.sync_copy(
        zeros,
        acc_sh.at[pl.ds(s * ROWS_PER_TILE, ROWS_PER_TILE)],
    )

    cvec = jnp.full((16,), c * N_NODES, jnp.int32)

    def off_body(j, carry):
        for g in range(CHUNK // 16):
            sl = pl.ds(g * 16, 16)
            idx_v[j, 0, sl] = idx_v[j, 0, sl] + cvec
        return carry

    lax.fori_loop(0, NCH, off_body, 0)

    rows_bufs = (rows0_v, rows1_v)
    gsems = (sg0, sg1)
    ssems = (ss0, ss1)

    def g_start(j, b):
        pltpu.async_copy(h2.at[idx_v.at[j, 0]], rows_bufs[b], gsems[b])

    def g_wait(j, b):
        pltpu.make_async_copy(
            h2.at[idx_v.at[j, 0]], rows_bufs[b], gsems[b]).wait()

    def s_start(j, b):
        pltpu.async_copy(
            rows_bufs[b], acc_sh.at[idx_v.at[j, 1]], ssems[b], add=True)

    def s_wait(j, b):
        pltpu.make_async_copy(
            rows_bufs[b], acc_sh.at[idx_v.at[j, 1]], ssems[b]).wait()

    def scale(j, b):
        rows = rows_bufs[b]

        def g_body(g, carry2):
            gbase = pl.multiple_of(g * 16, 16)
            norm16 = norms_v[j, pl.ds(gbase, 16)]
            nbs = [jnp.full((16,), norm16[jj], jnp.float32)
                   for jj in range(16)]
            for k in range(DH // 16):
                sl = pl.ds(k * 16, 16)
                ts = [rows[gbase + jj, sl] for jj in range(16)]
                rs = [t * nb for t, nb in zip(ts, nbs)]
                for jj in range(16):
                    rows[gbase + jj, sl] = rs[jj]
            return carry2

        lax.fori_loop(0, CHUNK // 16, g_body, 0)

    plsc.subcore_barrier()
    g_start(0, 0)
    g_start(1, 1)

    def pair_body(jp, carry):
        j0 = jp * 2
        j1 = j0 + 1
        g_wait(j0, 0)
        scale(j0, 0)
        s_start(j0, 0)
        g_wait(j1, 1)
        scale(j1, 1)
        s_start(j1, 1)
        s_wait(j0, 0)

        @pl.when(j0 + 2 < NCH)
        def _refill0():
            g_start(j0 + 2, 0)

        s_wait(j1, 1)

        @pl.when(j1 + 2 < NCH)
        def _refill1():
            g_start(j1 + 2, 1)

        return carry

    lax.fori_loop(0, NCH // 2, pair_body, 0)
    plsc.subcore_barrier()

    @pl.when(s < N_TILES - 1)
    def _copy_full():
        r0 = pl.multiple_of(s * ROWS_PER_TILE, 8)
        pltpu.sync_copy(
            acc_sh.at[pl.ds(r0, ROWS_PER_TILE)],
            out.at[c, pl.ds(r0, ROWS_PER_TILE)],
        )

    @pl.when(s == N_TILES - 1)
    def _copy_tail():
        r0 = (N_TILES - 1) * ROWS_PER_TILE
        tail = N_NODES - r0
        pltpu.sync_copy(
            acc_sh.at[pl.ds(r0, tail)],
            out.at[c, pl.ds(r0, tail)],
        )


_mp = functools.partial(
    pl.kernel,
    mesh=plsc.VectorSubcoreMesh(core_axis_name="c", subcore_axis_name="s"),
    compiler_params=pltpu.CompilerParams(use_tc_tiling_on_sc=False),
    out_type=jax.ShapeDtypeStruct((2, N_NODES, DH), jnp.float32),
    scratch_types=[
        pltpu.VMEM((NCH, 2, CHUNK), jnp.int32),
        pltpu.VMEM((NCH, CHUNK), jnp.float32),
        pltpu.VMEM((CHUNK, DH), jnp.float32),
        pltpu.VMEM((CHUNK, DH), jnp.float32),
        pltpu.VMEM_SHARED((N_PAD, DH), jnp.float32),
        pltpu.SemaphoreType.DMA,
        pltpu.SemaphoreType.DMA,
        pltpu.SemaphoreType.DMA,
        pltpu.SemaphoreType.DMA,
    ],
)(_mp_body)


def kernel(x, edge_index, norm, W, b, gamma, beta):
    x2 = x.reshape(N_NODES, D_IN)
    wt = W.T
    b2 = b.reshape(1, D_OUT)
    g2 = gamma.reshape(N_NODES, 1)
    be2 = beta.reshape(N_NODES, 1)
    h2 = _dense(x2, wt, b2, g2, be2).reshape(2 * N_NODES, DH)

    ei = edge_index.astype(jnp.int32)
    pad = E_PAD - N_EDGES
    src = jnp.pad(ei[1], (0, pad))
    dst = jnp.pad(ei[0], (0, pad))
    nrm = jnp.pad(norm.reshape(N_EDGES), (0, pad))
    idxpack = jnp.stack(
        [src.reshape(N_TILES, NCH, CHUNK),
         dst.reshape(N_TILES, NCH, CHUNK)], axis=2)
    normpack = nrm.reshape(N_TILES, NCH, CHUNK)

    zeros = jnp.zeros((ROWS_PER_TILE, DH), jnp.float32)
    out = _mp(h2, idxpack, normpack, zeros)
    return jnp.concatenate([out[0], out[1]], axis=-1).reshape(
        1, N_NODES, D_OUT)


# speedup vs baseline: 1.2120x; 1.2120x over previous
import functools
import math

import jax
import jax.numpy as jnp
from jax import lax
from jax.experimental import pallas as pl
from jax.experimental.pallas import tpu as pltpu
from jax.experimental.pallas import tpu_sc as plsc

N_NODES = 10000
D_IN = 128
D_OUT = 128
DH = 64
N_EDGES = 320000
EPS = 1e-5

N_TILES = 16
CHUNK = 128
NCH = 160
E_PAD = NCH * N_TILES * CHUNK
EDGES_PER_TILE = NCH * CHUNK

BN = 1000
N_PAD = 10240
ROWS_PER_TILE = N_PAD // N_TILES

_INV_SQRT2 = 1.0 / math.sqrt(2.0)


def _dense_body(x_ref, wt_ref, b_ref, g_ref, be_ref, o_ref):
    h = lax.dot_general(
        x_ref[...], wt_ref[...], (((1,), (0,)), ((), ())),
        preferred_element_type=jnp.float32,
    )
    h = h + b_ref[...]
    m = jnp.mean(h, axis=1, keepdims=True)
    d = h - m
    v = jnp.mean(d * d, axis=1, keepdims=True)
    hn = d * lax.rsqrt(v + EPS)
    hn = hn * g_ref[...] + be_ref[...]
    g = 0.5 * hn * (1.0 + lax.erf(hn * _INV_SQRT2))
    o_ref[0] = g[:, :DH]
    o_ref[1] = g[:, DH:]


def _dense(x2, wt, b2, g2, be2):
    return pl.pallas_call(
        _dense_body,
        grid=(N_NODES // BN,),
        in_specs=[
            pl.BlockSpec((BN, D_IN), lambda i: (i, 0)),
            pl.BlockSpec((D_IN, D_OUT), lambda i: (0, 0)),
            pl.BlockSpec((1, D_OUT), lambda i: (0, 0)),
            pl.BlockSpec((BN, 1), lambda i: (i, 0)),
            pl.BlockSpec((BN, 1), lambda i: (i, 0)),
        ],
        out_specs=pl.BlockSpec((2, BN, DH), lambda i: (0, i, 0)),
        out_shape=jax.ShapeDtypeStruct((2, N_NODES, DH), jnp.float32),
    )(x2, wt, b2, g2, be2)


def _mp_body(h2, idxpack, normpack, zeros, out,
             idxr_v, normr_v, rows0_v, rows1_v, h_sh, acc_sh,
             si0, si1, si2, si3, sg0, sg1, ss0, ss1):
    c = lax.axis_index("c")
    s = lax.axis_index("s")

    isems = (si0, si1, si2, si3)
    rows_bufs = (rows0_v, rows1_v)
    gsems = (sg0, sg1)
    ssems = (ss0, ss1)

    def i_start(j, q):
        pltpu.async_copy(idxpack.at[s, j], idxr_v.at[q], isems[q])
        pltpu.async_copy(normpack.at[s, j], normr_v.at[q], isems[q])

    def i_wait(j, q):
        pltpu.make_async_copy(
            idxpack.at[s, j], idxr_v.at[q], isems[q]).wait()
        pltpu.make_async_copy(
            normpack.at[s, j], normr_v.at[q], isems[q]).wait()

    def g_start(q, b):
        pltpu.async_copy(h_sh.at[idxr_v.at[q, 0]], rows_bufs[b], gsems[b])

    def g_wait(q, b):
        pltpu.make_async_copy(
            h_sh.at[idxr_v.at[q, 0]], rows_bufs[b], gsems[b]).wait()

    def s_start(q, b):
        pltpu.async_copy(
            rows_bufs[b], acc_sh.at[idxr_v.at[q, 1]], ssems[b], add=True)

    def s_wait(q, b):
        pltpu.make_async_copy(
            rows_bufs[b], acc_sh.at[idxr_v.at[q, 1]], ssems[b]).wait()

    def scale(q, b):
        rows = rows_bufs[b]

        def g_body(g, carry2):
            gbase = pl.multiple_of(g * 16, 16)
            norm16 = normr_v[q, pl.ds(gbase, 16)]
            nbs = [jnp.full((16,), norm16[jj], jnp.float32)
                   for jj in range(16)]
            for k in range(DH // 16):
                sl = pl.ds(k * 16, 16)
                ts = [rows[gbase + jj, sl] for jj in range(16)]
                rs = [t * nb for t, nb in zip(ts, nbs)]
                for jj in range(16):
                    rows[gbase + jj, sl] = rs[jj]
            return carry2

        lax.fori_loop(0, CHUNK // 16, g_body, 0)

    pltpu.sync_copy(
        zeros,
        acc_sh.at[pl.ds(s * ROWS_PER_TILE, ROWS_PER_TILE)],
    )

    @pl.when(s < N_TILES - 1)
    def _stage_full():
        r0 = pl.multiple_of(s * ROWS_PER_TILE, 8)
        pltpu.sync_copy(
            h2.at[pl.ds(pl.multiple_of(c * N_NODES + r0, 8), ROWS_PER_TILE)],
            h_sh.at[pl.ds(r0, ROWS_PER_TILE)],
        )

    @pl.when(s == N_TILES - 1)
    def _stage_tail():
        r0 = (N_TILES - 1) * ROWS_PER_TILE
        tail = N_NODES - r0
        pltpu.sync_copy(
            h2.at[pl.ds(pl.multiple_of(c * N_NODES + r0, 8), tail)],
            h_sh.at[pl.ds(r0, tail)],
        )

    for q in range(4):
        i_start(q, q)
    plsc.subcore_barrier()
    i_wait(0, 0)
    g_start(0, 0)
    i_wait(1, 1)
    g_start(1, 1)

    def quad_body(jq, carry):
        j0 = jq * 4
        g_wait(0, 0)
        scale(0, 0)
        s_start(0, 0)
        g_wait(1, 1)
        scale(1, 1)
        s_start(1, 1)
        s_wait(0, 0)
        i_wait(j0 + 2, 2)
        g_start(2, 0)

        @pl.when(j0 + 4 < NCH)
        def _pf0():
            i_start(j0 + 4, 0)
        g_wait(2, 0)
        scale(2, 0)
        s_start(2, 0)
        s_wait(1, 1)
        i_wait(j0 + 3, 3)
        g_start(3, 1)

        @pl.when(j0 + 5 < NCH)
        def _pf1():
            i_start(j0 + 5, 1)
        g_wait(3, 1)
        scale(3, 1)
        s_start(3, 1)
        s_wait(2, 0)

        @pl.when(j0 + 4 < NCH)
        def _next0():
            i_wait(j0 + 4, 0)
            g_start(0, 0)
            i_start(j0 + 6, 2)

        s_wait(3, 1)

        @pl.when(j0 + 5 < NCH)
        def _next1():
            i_wait(j0 + 5, 1)
            g_start(1, 1)
            i_start(j0 + 7, 3)

        return carry

    lax.fori_loop(0, NCH // 4, quad_body, 0)
    plsc.subcore_barrier()

    @pl.when(s < N_TILES - 1)
    def _copy_full():
        r0 = pl.multiple_of(s * ROWS_PER_TILE, 8)
        pltpu.sync_copy(
            acc_sh.at[pl.ds(r0, ROWS_PER_TILE)],
            out.at[c, pl.ds(r0, ROWS_PER_TILE)],
        )

    @pl.when(s == N_TILES - 1)
    def _copy_tail():
        r0 = (N_TILES - 1) * ROWS_PER_TILE
        tail = N_NODES - r0
        pltpu.sync_copy(
            acc_sh.at[pl.ds(r0, tail)],
            out.at[c, pl.ds(r0, tail)],
        )


_mp = functools.partial(
    pl.kernel,
    mesh=plsc.VectorSubcoreMesh(core_axis_name="c", subcore_axis_name="s"),
    compiler_params=pltpu.CompilerParams(use_tc_tiling_on_sc=False),
    out_type=jax.ShapeDtypeStruct((2, N_NODES, DH), jnp.float32),
    scratch_types=[
        pltpu.VMEM((4, 2, CHUNK), jnp.int32),
        pltpu.VMEM((4, CHUNK), jnp.float32),
        pltpu.VMEM((CHUNK, DH), jnp.float32),
        pltpu.VMEM((CHUNK, DH), jnp.float32),
        pltpu.VMEM_SHARED((N_PAD, DH), jnp.float32),
        pltpu.VMEM_SHARED((N_PAD, DH), jnp.float32),
        pltpu.SemaphoreType.DMA,
        pltpu.SemaphoreType.DMA,
        pltpu.SemaphoreType.DMA,
        pltpu.SemaphoreType.DMA,
        pltpu.SemaphoreType.DMA,
        pltpu.SemaphoreType.DMA,
        pltpu.SemaphoreType.DMA,
        pltpu.SemaphoreType.DMA,
    ],
)(_mp_body)


def kernel(x, edge_index, norm, W, b, gamma, beta):
    x2 = x.reshape(N_NODES, D_IN)
    wt = W.T
    b2 = b.reshape(1, D_OUT)
    g2 = gamma.reshape(N_NODES, 1)
    be2 = beta.reshape(N_NODES, 1)
    h2 = _dense(x2, wt, b2, g2, be2).reshape(2 * N_NODES, DH)

    ei = edge_index.astype(jnp.int32)
    pad = E_PAD - N_EDGES
    src = jnp.pad(ei[1], (0, pad))
    dst = jnp.pad(ei[0], (0, pad))
    nrm = jnp.pad(norm.reshape(N_EDGES), (0, pad))
    idxpack = jnp.stack(
        [src.reshape(N_TILES, NCH, CHUNK),
         dst.reshape(N_TILES, NCH, CHUNK)], axis=2)
    normpack = nrm.reshape(N_TILES, NCH, CHUNK)

    zeros = jnp.zeros((ROWS_PER_TILE, DH), jnp.float32)
    out = _mp(h2, idxpack, normpack, zeros)
    return jnp.concatenate([out[0], out[1]], axis=-1).reshape(
        1, N_NODES, D_OUT)


# speedup vs baseline: 1.3817x; 1.1400x over previous
import functools
import math

import jax
import jax.numpy as jnp
from jax import lax
from jax.experimental import pallas as pl
from jax.experimental.pallas import tpu as pltpu
from jax.experimental.pallas import tpu_sc as plsc

N_NODES = 10000
D_IN = 128
D_OUT = 128
DH = 64
N_EDGES = 320000
EPS = 1e-5

N_TILES = 16
CHUNK = 128
NCH = 160
E_PAD = NCH * N_TILES * CHUNK
EDGES_PER_TILE = NCH * CHUNK

BN = 1000
N_PAD = 10240
ROWS_PER_TILE = N_PAD // N_TILES

_INV_SQRT2 = 1.0 / math.sqrt(2.0)


def _dense_body(x_ref, wt_ref, b_ref, g_ref, be_ref, o_ref):
    h = lax.dot_general(
        x_ref[...], wt_ref[...], (((1,), (0,)), ((), ())),
        preferred_element_type=jnp.float32,
    )
    h = h + b_ref[...]
    m = jnp.mean(h, axis=1, keepdims=True)
    d = h - m
    v = jnp.mean(d * d, axis=1, keepdims=True)
    hn = d * lax.rsqrt(v + EPS)
    hn = hn * g_ref[...] + be_ref[...]
    g = 0.5 * hn * (1.0 + lax.erf(hn * _INV_SQRT2))
    o_ref[0] = g[:, :DH]
    o_ref[1] = g[:, DH:]


def _dense(x2, wt, b2, g2, be2):
    return pl.pallas_call(
        _dense_body,
        grid=(N_NODES // BN,),
        in_specs=[
            pl.BlockSpec((BN, D_IN), lambda i: (i, 0)),
            pl.BlockSpec((D_IN, D_OUT), lambda i: (0, 0)),
            pl.BlockSpec((1, D_OUT), lambda i: (0, 0)),
            pl.BlockSpec((BN, 1), lambda i: (i, 0)),
            pl.BlockSpec((BN, 1), lambda i: (i, 0)),
        ],
        out_specs=pl.BlockSpec((2, BN, DH), lambda i: (0, i, 0)),
        out_shape=jax.ShapeDtypeStruct((2, N_NODES, DH), jnp.float32),
    )(x2, wt, b2, g2, be2)


def _mp_body(h2, idxpack, normpack, zeros, out,
             idxr_v, normr_v, rows0_v, rows1_v, rows2_v, rows3_v,
             h_sh, acc_sh,
             si0, si1, si2, si3, si4, si5, si6, si7,
             sg0, sg1, sg2, sg3, ss0, ss1, ss2, ss3):
    c = lax.axis_index("c")
    s = lax.axis_index("s")

    isems = (si0, si1, si2, si3, si4, si5, si6, si7)
    rows_bufs = (rows0_v, rows1_v, rows2_v, rows3_v)
    gsems = (sg0, sg1, sg2, sg3)
    ssems = (ss0, ss1, ss2, ss3)

    def i_start(j, q):
        pltpu.async_copy(idxpack.at[s, j], idxr_v.at[q], isems[q])
        pltpu.async_copy(normpack.at[s, j], normr_v.at[q], isems[q])

    def i_wait(j, q):
        pltpu.make_async_copy(
            idxpack.at[s, j], idxr_v.at[q], isems[q]).wait()
        pltpu.make_async_copy(
            normpack.at[s, j], normr_v.at[q], isems[q]).wait()

    def g_start(q, b):
        pltpu.async_copy(h_sh.at[idxr_v.at[q, 0]], rows_bufs[b], gsems[b])

    def g_wait(q, b):
        pltpu.make_async_copy(
            h_sh.at[idxr_v.at[q, 0]], rows_bufs[b], gsems[b]).wait()

    def s_start(q, b):
        pltpu.async_copy(
            rows_bufs[b], acc_sh.at[idxr_v.at[q, 1]], ssems[b], add=True)

    def s_wait(q, b):
        pltpu.make_async_copy(
            rows_bufs[b], acc_sh.at[idxr_v.at[q, 1]], ssems[b]).wait()

    def scale(q, b):
        rows = rows_bufs[b]

        def g_body(g, carry2):
            gbase = pl.multiple_of(g * 16, 16)
            norm16 = normr_v[q, pl.ds(gbase, 16)]
            nbs = [jnp.full((16,), norm16[jj], jnp.float32)
                   for jj in range(16)]
            for k in range(DH // 16):
                sl = pl.ds(k * 16, 16)
                ts = [rows[gbase + jj, sl] for jj in range(16)]
                rs = [t * nb for t, nb in zip(ts, nbs)]
                for jj in range(16):
                    rows[gbase + jj, sl] = rs[jj]
            return carry2

        lax.fori_loop(0, CHUNK // 16, g_body, 0)

    pltpu.sync_copy(
        zeros,
        acc_sh.at[pl.ds(s * ROWS_PER_TILE, ROWS_PER_TILE)],
    )

    @pl.when(s < N_TILES - 1)
    def _stage_full():
        r0 = pl.multiple_of(s * ROWS_PER_TILE, 8)
        pltpu.sync_copy(
            h2.at[pl.ds(pl.multiple_of(c * N_NODES + r0, 8), ROWS_PER_TILE)],
            h_sh.at[pl.ds(r0, ROWS_PER_TILE)],
        )

    @pl.when(s == N_TILES - 1)
    def _stage_tail():
        r0 = (N_TILES - 1) * ROWS_PER_TILE
        tail = N_NODES - r0
        pltpu.sync_copy(
            h2.at[pl.ds(pl.multiple_of(c * N_NODES + r0, 8), tail)],
            h_sh.at[pl.ds(r0, tail)],
        )

    for q in range(8):
        i_start(q, q)
    plsc.subcore_barrier()
    i_wait(0, 0)
    g_start(0, 0)
    i_wait(1, 1)
    g_start(1, 1)

    def oct_body(jo, carry):
        j0 = jo * 8
        for k in range(8):
            b = k % 4
            q = k
            g_wait(q, b)
            scale(q, b)
            s_start(q, b)
            qn = (k + 2) % 8
            bn = (k + 2) % 4
            qp = (k - 2) % 8
            if k < 2:
                @pl.when(j0 + k - 2 >= 0)
                def _drain(qp=qp, bn=bn):
                    s_wait(qp, bn)
            else:
                s_wait(qp, bn)

            @pl.when(j0 + k + 2 < NCH)
            def _refill(k=k, qn=qn, bn=bn):
                i_wait(j0 + k + 2, qn)
                g_start(qn, bn)

            @pl.when(j0 + k + 6 < NCH)
            def _prefetch(k=k, qf=(k + 6) % 8):
                i_start(j0 + k + 6, qf)
        return carry

    lax.fori_loop(0, NCH // 8, oct_body, 0)
    s_wait(6, 2)
    s_wait(7, 3)
    plsc.subcore_barrier()

    @pl.when(s < N_TILES - 1)
    def _copy_full():
        r0 = pl.multiple_of(s * ROWS_PER_TILE, 8)
        pltpu.sync_copy(
            acc_sh.at[pl.ds(r0, ROWS_PER_TILE)],
            out.at[c, pl.ds(r0, ROWS_PER_TILE)],
        )

    @pl.when(s == N_TILES - 1)
    def _copy_tail():
        r0 = (N_TILES - 1) * ROWS_PER_TILE
        tail = N_NODES - r0
        pltpu.sync_copy(
            acc_sh.at[pl.ds(r0, tail)],
            out.at[c, pl.ds(r0, tail)],
        )


_mp = functools.partial(
    pl.kernel,
    mesh=plsc.VectorSubcoreMesh(core_axis_name="c", subcore_axis_name="s"),
    compiler_params=pltpu.CompilerParams(use_tc_tiling_on_sc=False),
    out_type=jax.ShapeDtypeStruct((2, N_NODES, DH), jnp.float32),
    scratch_types=[
        pltpu.VMEM((8, 2, CHUNK), jnp.int32),
        pltpu.VMEM((8, CHUNK), jnp.float32),
        pltpu.VMEM((CHUNK, DH), jnp.float32),
        pltpu.VMEM((CHUNK, DH), jnp.float32),
        pltpu.VMEM((CHUNK, DH), jnp.float32),
        pltpu.VMEM((CHUNK, DH), jnp.float32),
        pltpu.VMEM_SHARED((N_PAD, DH), jnp.float32),
        pltpu.VMEM_SHARED((N_PAD, DH), jnp.float32),
    ] + [pltpu.SemaphoreType.DMA] * 16,
)(_mp_body)


def kernel(x, edge_index, norm, W, b, gamma, beta):
    x2 = x.reshape(N_NODES, D_IN)
    wt = W.T
    b2 = b.reshape(1, D_OUT)
    g2 = gamma.reshape(N_NODES, 1)
    be2 = beta.reshape(N_NODES, 1)
    h2 = _dense(x2, wt, b2, g2, be2).reshape(2 * N_NODES, DH)

    ei = edge_index.astype(jnp.int32)
    pad = E_PAD - N_EDGES
    src = jnp.pad(ei[1], (0, pad))
    dst = jnp.pad(ei[0], (0, pad))
    nrm = jnp.pad(norm.reshape(N_EDGES), (0, pad))
    idxpack = jnp.stack(
        [src.reshape(N_TILES, NCH, CHUNK),
         dst.reshape(N_TILES, NCH, CHUNK)], axis=2)
    normpack = nrm.reshape(N_TILES, NCH, CHUNK)

    zeros = jnp.zeros((ROWS_PER_TILE, DH), jnp.float32)
    out = _mp(h2, idxpack, normpack, zeros)
    return jnp.concatenate([out[0], out[1]], axis=-1).reshape(
        1, N_NODES, D_OUT)


# speedup vs baseline: 1.4776x; 1.0694x over previous
import functools
import math

import jax
import jax.numpy as jnp
from jax import lax
from jax.experimental import pallas as pl
from jax.experimental.pallas import tpu as pltpu
from jax.experimental.pallas import tpu_sc as plsc

N_NODES = 10000
D_IN = 128
D_OUT = 128
DH = 64
N_EDGES = 320000
EPS = 1e-5

N_TILES = 16
CHUNK = 128
NCH = 160
E_PAD = NCH * N_TILES * CHUNK
EDGES_PER_TILE = NCH * CHUNK

BN = 1000
N_PAD = 10240
ROWS_PER_TILE = N_PAD // N_TILES

_INV_SQRT2 = 1.0 / math.sqrt(2.0)


def _dense_body(x_ref, wt_ref, b_ref, g_ref, be_ref, o_ref):
    h = lax.dot_general(
        x_ref[...], wt_ref[...], (((1,), (0,)), ((), ())),
        preferred_element_type=jnp.float32,
    )
    h = h + b_ref[...]
    m = jnp.mean(h, axis=1, keepdims=True)
    d = h - m
    v = jnp.mean(d * d, axis=1, keepdims=True)
    hn = d * lax.rsqrt(v + EPS)
    hn = hn * g_ref[...] + be_ref[...]
    g = 0.5 * hn * (1.0 + lax.erf(hn * _INV_SQRT2))
    o_ref[0] = g[:, :DH]
    o_ref[1] = g[:, DH:]


def _dense(x2, wt, b2, g2, be2):
    return pl.pallas_call(
        _dense_body,
        grid=(N_NODES // BN,),
        in_specs=[
            pl.BlockSpec((BN, D_IN), lambda i: (i, 0)),
            pl.BlockSpec((D_IN, D_OUT), lambda i: (0, 0)),
            pl.BlockSpec((1, D_OUT), lambda i: (0, 0)),
            pl.BlockSpec((BN, 1), lambda i: (i, 0)),
            pl.BlockSpec((BN, 1), lambda i: (i, 0)),
        ],
        out_specs=pl.BlockSpec((2, BN, DH), lambda i: (0, i, 0)),
        out_shape=jax.ShapeDtypeStruct((2, N_NODES, DH), jnp.float32),
    )(x2, wt, b2, g2, be2)


def _mp_body(h2, idxpack, normpack, zeros, out,
             idxr_v, normr_v, rows0_v, rows1_v, rows2_v, rows3_v,
             h_sh, acc_sh,
             si0, si1, si2, si3, si4, si5, si6, si7,
             sg0, sg1, sg2, sg3, ss0, ss1, ss2, ss3):
    c = lax.axis_index("c")
    s = lax.axis_index("s")

    isems = (si0, si1, si2, si3, si4, si5, si6, si7)
    rows_bufs = (rows0_v, rows1_v, rows2_v, rows3_v)
    gsems = (sg0, sg1, sg2, sg3)
    ssems = (ss0, ss1, ss2, ss3)

    def i_start(j, q):
        pltpu.async_copy(idxpack.at[s, j], idxr_v.at[q], isems[q])
        pltpu.async_copy(normpack.at[s, j], normr_v.at[q], isems[q])

    def i_wait(j, q):
        pltpu.make_async_copy(
            idxpack.at[s, j], idxr_v.at[q], isems[q]).wait()
        pltpu.make_async_copy(
            normpack.at[s, j], normr_v.at[q], isems[q]).wait()

    def g_start(q, b):
        pltpu.async_copy(h_sh.at[idxr_v.at[q, 0]], rows_bufs[b], gsems[b])

    def g_wait(q, b):
        pltpu.make_async_copy(
            h_sh.at[idxr_v.at[q, 0]], rows_bufs[b], gsems[b]).wait()

    def s_start(q, b):
        pltpu.async_copy(
            rows_bufs[b], acc_sh.at[idxr_v.at[q, 1]], ssems[b], add=True)

    def s_wait(q, b):
        pltpu.make_async_copy(
            rows_bufs[b], acc_sh.at[idxr_v.at[q, 1]], ssems[b]).wait()

    def scale(q, b):
        rows = rows_bufs[b]

        def g_body(g, carry2):
            gbase = pl.multiple_of(g * 16, 16)
            norm16 = normr_v[q, pl.ds(gbase, 16)]
            nbs = [jnp.full((16,), norm16[jj], jnp.float32)
                   for jj in range(16)]
            for k in range(DH // 16):
                sl = pl.ds(k * 16, 16)
                ts = [rows[gbase + jj, sl] for jj in range(16)]
                rs = [t * nb for t, nb in zip(ts, nbs)]
                for jj in range(16):
                    rows[gbase + jj, sl] = rs[jj]
            return carry2

        lax.fori_loop(0, CHUNK // 16, g_body, 0)

    pltpu.sync_copy(
        zeros,
        acc_sh.at[pl.ds(s * ROWS_PER_TILE, ROWS_PER_TILE)],
    )

    @pl.when(s < N_TILES - 1)
    def _stage_full():
        r0 = pl.multiple_of(s * ROWS_PER_TILE, 8)
        pltpu.sync_copy(
            h2.at[pl.ds(pl.multiple_of(c * N_NODES + r0, 8), ROWS_PER_TILE)],
            h_sh.at[pl.ds(r0, ROWS_PER_TILE)],
        )

    @pl.when(s == N_TILES - 1)
    def _stage_tail():
        r0 = (N_TILES - 1) * ROWS_PER_TILE
        tail = N_NODES - r0
        pltpu.sync_copy(
            h2.at[pl.ds(pl.multiple_of(c * N_NODES + r0, 8), tail)],
            h_sh.at[pl.ds(r0, tail)],
        )

    for q in range(8):
        i_start(q, q)
    plsc.subcore_barrier()
    i_wait(0, 0)
    g_start(0, 0)
    i_wait(1, 1)
    g_start(1, 1)

    def oct_body(jo, carry):
        j0 = jo * 8
        for k in range(8):
            b = k % 4
            q = k
            g_wait(q, b)
            scale(q, b)
            s_start(q, b)
            qn = (k + 2) % 8
            bn = (k + 2) % 4
            qp = (k - 2) % 8
            if k < 2:
                @pl.when(j0 + k - 2 >= 0)
                def _drain(qp=qp, bn=bn):
                    s_wait(qp, bn)
            else:
                s_wait(qp, bn)

            @pl.when(j0 + k + 2 < NCH)
            def _refill(k=k, qn=qn, bn=bn):
                i_wait(j0 + k + 2, qn)
                g_start(qn, bn)

            @pl.when(j0 + k + 6 < NCH)
            def _prefetch(k=k, qf=(k + 6) % 8):
                i_start(j0 + k + 6, qf)
        return carry

    lax.fori_loop(0, NCH // 8, oct_body, 0)
    s_wait(6, 2)
    s_wait(7, 3)
    plsc.subcore_barrier()

    @pl.when(s < N_TILES - 1)
    def _copy_full():
        r0 = pl.multiple_of(s * ROWS_PER_TILE, 8)
        pltpu.sync_copy(
            acc_sh.at[pl.ds(r0, ROWS_PER_TILE)],
            out.at[pl.ds(r0, ROWS_PER_TILE), pl.ds(c * DH, DH)],
        )

    @pl.when(s == N_TILES - 1)
    def _copy_tail():
        r0 = (N_TILES - 1) * ROWS_PER_TILE
        tail = N_NODES - r0
        pltpu.sync_copy(
            acc_sh.at[pl.ds(r0, tail)],
            out.at[pl.ds(r0, tail), pl.ds(c * DH, DH)],
        )


_mp = functools.partial(
    pl.kernel,
    mesh=plsc.VectorSubcoreMesh(core_axis_name="c", subcore_axis_name="s"),
    compiler_params=pltpu.CompilerParams(use_tc_tiling_on_sc=False),
    out_type=jax.ShapeDtypeStruct((N_NODES, D_OUT), jnp.float32),
    scratch_types=[
        pltpu.VMEM((8, 2, CHUNK), jnp.int32),
        pltpu.VMEM((8, CHUNK), jnp.float32),
        pltpu.VMEM((CHUNK, DH), jnp.float32),
        pltpu.VMEM((CHUNK, DH), jnp.float32),
        pltpu.VMEM((CHUNK, DH), jnp.float32),
        pltpu.VMEM((CHUNK, DH), jnp.float32),
        pltpu.VMEM_SHARED((N_PAD, DH), jnp.float32),
        pltpu.VMEM_SHARED((N_PAD, DH), jnp.float32),
    ] + [pltpu.SemaphoreType.DMA] * 16,
)(_mp_body)


def kernel(x, edge_index, norm, W, b, gamma, beta):
    x2 = x.reshape(N_NODES, D_IN)
    wt = W.T
    b2 = b.reshape(1, D_OUT)
    g2 = gamma.reshape(N_NODES, 1)
    be2 = beta.reshape(N_NODES, 1)
    h2 = _dense(x2, wt, b2, g2, be2).reshape(2 * N_NODES, DH)

    ei = edge_index.astype(jnp.int32)
    pad = E_PAD - N_EDGES
    src = jnp.pad(ei[1], (0, pad))
    dst = jnp.pad(ei[0], (0, pad))
    nrm = jnp.pad(norm.reshape(N_EDGES), (0, pad))
    idxpack = jnp.stack(
        [src.reshape(N_TILES, NCH, CHUNK),
         dst.reshape(N_TILES, NCH, CHUNK)], axis=2)
    normpack = nrm.reshape(N_TILES, NCH, CHUNK)

    zeros = jnp.zeros((ROWS_PER_TILE, DH), jnp.float32)
    out = _mp(h2, idxpack, normpack, zeros)
    return out.reshape(1, N_NODES, D_OUT)
